# baseline (speedup 1.0000x reference)
import functools

import jax
import jax.numpy as jnp
from jax import lax
from jax.experimental import pallas as pl
from jax.experimental.pallas import tpu as pltpu
from jax.experimental.pallas import tpu_sc as plsc

NUM_CORES = 2
NUM_SUBCORES = 16
LANES = 16
NW = NUM_CORES * NUM_SUBCORES
R = 8
NBUF = 4


def _sc_body(D, b_per_w, tab_n, emb_hbm, idx_hbm, table_hbm, out_hbm,
             table_sp, table_v, idx_v, ebuf0, ebuf1, ebuf2, ebuf3,
             tab_sem, in_sem0, in_sem1, in_sem2, in_sem3,
             out_sem0, out_sem1, out_sem2, out_sem3):
    c = lax.axis_index("c")
    s = lax.axis_index("s")
    wid = s * NUM_CORES + c
    base = wid * b_per_w
    n_chunks = b_per_w // R

    ebufs = (ebuf0, ebuf1, ebuf2, ebuf3)
    in_sems = (in_sem0, in_sem1, in_sem2, in_sem3)
    out_sems = (out_sem0, out_sem1, out_sem2, out_sem3)

    def start_in(j, b):
        pltpu.async_copy(emb_hbm.at[pl.ds(base + j * R, R)], ebufs[b],
                         in_sems[b])

    def wait_in(j, b):
        pltpu.make_async_copy(emb_hbm.at[pl.ds(base + j * R, R)], ebufs[b],
                              in_sems[b]).wait()

    def start_out(j, b):
        pltpu.async_copy(ebufs[b], out_hbm.at[pl.ds(base + j * R, R)],
                         out_sems[b])

    def wait_out(j, b):
        pltpu.make_async_copy(ebufs[b], out_hbm.at[pl.ds(base + j * R, R)],
                              out_sems[b]).wait()

    start_in(0, 0)

    pltpu.sync_copy(table_hbm.at[s], table_sp.at[s])
    plsc.subcore_barrier()
    for k in range(NUM_SUBCORES):
        pltpu.async_copy(table_sp.at[k],
                         table_v.at[pl.ds(k * tab_n, tab_n)], tab_sem)

    pltpu.sync_copy(idx_hbm.at[wid], idx_v.at[pl.ds(0, b_per_w)])
    for k in range(NUM_SUBCORES):
        pltpu.make_async_copy(table_sp.at[k],
                              table_v.at[pl.ds(k * tab_n, tab_n)],
                              tab_sem).wait()

    def chunk_group(g, carry):
        for bs in range(NBUF):
            j = g + bs

            wait_in(j, bs)

            @pl.when(j + 1 < n_chunks)
            def _prefetch():
                nb = (bs + 1) % NBUF

                @pl.when(j >= NBUF - 1)
                def _free():
                    wait_out(j - (NBUF - 1), nb)

                start_in(j + 1, nb)

            @plsc.parallel_loop(0, R, 1, unroll=1)
            def _add_row(r):
                iv = idx_v[pl.ds(j * R + r, LANES)]
                rbase = iv[0] * D
                for cc in range(D // LANES):
                    v = table_v[pl.ds(rbase + cc * LANES, LANES)]
                    plsc.addupdate(
                        ebufs[bs].at[r, pl.ds(cc * LANES, LANES)], v)

            plsc.subcore_barrier()
            start_out(j, bs)
        return carry

    lax.fori_loop(0, n_chunks // NBUF, lambda t, cr: chunk_group(t * NBUF, cr),
                  None)

    for j in range(n_chunks - NBUF, n_chunks):
        wait_out(j, j % NBUF)


def kernel(batch_Phrase_emb, Phrase_type_ids, phrase_attribute_emb_all):
    B, D = batch_Phrase_emb.shape
    V = phrase_attribute_emb_all.shape[0]
    b_per_w = B // NW

    idx = Phrase_type_ids.astype(jnp.int32).reshape(NW, b_per_w)
    chunk_words = 128 * NUM_SUBCORES
    vd_pad = ((V * D + chunk_words - 1) // chunk_words) * chunk_words
    tab_n = vd_pad // NUM_SUBCORES
    table_flat = jnp.concatenate(
        [phrase_attribute_emb_all.reshape(V * D),
         jnp.zeros((vd_pad - V * D,), jnp.float32)])
    table_slices = table_flat.reshape(NUM_SUBCORES, tab_n)

    mesh = plsc.VectorSubcoreMesh(
        core_axis_name="c", subcore_axis_name="s",
        num_cores=NUM_CORES, num_subcores=NUM_SUBCORES)
    f = pl.kernel(
        functools.partial(_sc_body, D, b_per_w, tab_n),
        out_type=jax.ShapeDtypeStruct((B, D), jnp.float32),
        mesh=mesh,
        scratch_types=[
            pltpu.VMEM_SHARED((NUM_SUBCORES, tab_n), jnp.float32),
            pltpu.VMEM((NUM_SUBCORES * tab_n,), jnp.float32),
            pltpu.VMEM((b_per_w + LANES,), jnp.int32),
            pltpu.VMEM((R, D), jnp.float32),
            pltpu.VMEM((R, D), jnp.float32),
            pltpu.VMEM((R, D), jnp.float32),
            pltpu.VMEM((R, D), jnp.float32),
            pltpu.SemaphoreType.DMA,
            pltpu.SemaphoreType.DMA,
            pltpu.SemaphoreType.DMA,
            pltpu.SemaphoreType.DMA,
            pltpu.SemaphoreType.DMA,
            pltpu.SemaphoreType.DMA,
            pltpu.SemaphoreType.DMA,
            pltpu.SemaphoreType.DMA,
            pltpu.SemaphoreType.DMA,
        ],
    )
    return f(batch_Phrase_emb, idx, table_slices)


# speedup vs baseline: 1.3204x; 1.3204x over previous
import functools

import jax
import jax.numpy as jnp
from jax import lax
from jax.experimental import pallas as pl
from jax.experimental.pallas import tpu as pltpu
from jax.experimental.pallas import tpu_sc as plsc

NUM_CORES = 2
NUM_SUBCORES = 16
LANES = 16
NW = NUM_CORES * NUM_SUBCORES
R = 16
NBUF = 4
SC_ROWS = 4096
TC_BLK = 512
TABLE_PAD = 128


def _sc_body(D, b_per_w, row0, emb_hbm, idx_hbm, table_hbm, out_hbm,
             table_v, idx_v, ebuf0, ebuf1, ebuf2, ebuf3,
             in_sem0, in_sem1, in_sem2, in_sem3,
             out_sem0, out_sem1, out_sem2, out_sem3):
    c = lax.axis_index("c")
    s = lax.axis_index("s")
    wid = s * NUM_CORES + c
    base_in = row0 + wid * b_per_w
    base_out = wid * b_per_w
    n_chunks = b_per_w // R

    ebufs = (ebuf0, ebuf1, ebuf2, ebuf3)
    in_sems = (in_sem0, in_sem1, in_sem2, in_sem3)
    out_sems = (out_sem0, out_sem1, out_sem2, out_sem3)

    pltpu.sync_copy(table_hbm, table_v)
    pltpu.sync_copy(idx_hbm.at[wid], idx_v.at[pl.ds(0, b_per_w)])

    def start_in(j, b):
        pltpu.async_copy(emb_hbm.at[pl.ds(base_in + j * R, R)], ebufs[b],
                         in_sems[b])

    def wait_in(j, b):
        pltpu.make_async_copy(emb_hbm.at[pl.ds(base_in + j * R, R)],
                              ebufs[b], in_sems[b]).wait()

    def start_out(j, b):
        pltpu.async_copy(ebufs[b], out_hbm.at[pl.ds(base_out + j * R, R)],
                         out_sems[b])

    def wait_out(j, b):
        pltpu.make_async_copy(ebufs[b],
                              out_hbm.at[pl.ds(base_out + j * R, R)],
                              out_sems[b]).wait()

    start_in(0, 0)

    def chunk_group(g, carry):
        for bs in range(NBUF):
            j = g + bs

            wait_in(j, bs)

            @pl.when(j + 1 < n_chunks)
            def _prefetch():
                nb = (bs + 1) % NBUF

                @pl.when(j >= NBUF - 1)
                def _free():
                    wait_out(j - (NBUF - 1), nb)

                start_in(j + 1, nb)

            @plsc.parallel_loop(0, R, 1, unroll=1)
            def _add_row(r):
                iv = idx_v[pl.ds(j * R + r, LANES)]
                rbase = iv[0] * D
                for cc in range(D // LANES):
                    v = table_v[pl.ds(rbase + cc * LANES, LANES)]
                    plsc.addupdate(
                        ebufs[bs].at[r, pl.ds(cc * LANES, LANES)], v)

            plsc.subcore_barrier()
            start_out(j, bs)
        return carry

    lax.fori_loop(0, n_chunks // NBUF,
                  lambda t, cr: chunk_group(t * NBUF, cr), None)

    for j in range(n_chunks - NBUF, n_chunks):
        wait_out(j, j % NBUF)


def _sc_part(emb, ids, table, row0, sc_rows):
    D = emb.shape[1]
    V = table.shape[0]
    b_per_w = sc_rows // NW

    idx = ids.astype(jnp.int32).reshape(NW, b_per_w)
    table_flat = table.reshape(V * D)

    mesh = plsc.VectorSubcoreMesh(
        core_axis_name="c", subcore_axis_name="s",
        num_cores=NUM_CORES, num_subcores=NUM_SUBCORES)
    f = pl.kernel(
        functools.partial(_sc_body, D, b_per_w, row0),
        out_type=jax.ShapeDtypeStruct((sc_rows, D), jnp.float32),
        mesh=mesh,
        scratch_types=[
            pltpu.VMEM((V * D,), jnp.float32),
            pltpu.VMEM((b_per_w + LANES,), jnp.int32),
            pltpu.VMEM((R, D), jnp.float32),
            pltpu.VMEM((R, D), jnp.float32),
            pltpu.VMEM((R, D), jnp.float32),
            pltpu.VMEM((R, D), jnp.float32),
            pltpu.SemaphoreType.DMA,
            pltpu.SemaphoreType.DMA,
            pltpu.SemaphoreType.DMA,
            pltpu.SemaphoreType.DMA,
            pltpu.SemaphoreType.DMA,
            pltpu.SemaphoreType.DMA,
            pltpu.SemaphoreType.DMA,
            pltpu.SemaphoreType.DMA,
        ],
    )
    return f(emb, idx, table_flat)


def _tc_body(emb_ref, ids_ref, table_ref, out_ref):
    ids = ids_ref[0, 0, :]
    onehot = (ids[:, None] == lax.broadcasted_iota(jnp.int32, (1, TABLE_PAD),
                                                   1)).astype(jnp.float32)
    gathered = jax.lax.dot_general(
        onehot, table_ref[...],
        dimension_numbers=(((1,), (0,)), ((), ())),
        preferred_element_type=jnp.float32)
    out_ref[...] = emb_ref[...] + gathered


def _tc_part(emb, ids, table, tc_rows):
    B, D = emb.shape
    V = table.shape[0]
    table_pad = jnp.concatenate(
        [table, jnp.zeros((TABLE_PAD - V, D), jnp.float32)], axis=0)
    n_blk = tc_rows // TC_BLK
    ids_blk = ids.astype(jnp.int32).reshape(n_blk, 1, TC_BLK)
    return pl.pallas_call(
        _tc_body,
        out_shape=jax.ShapeDtypeStruct((B, D), jnp.float32),
        grid=(n_blk,),
        in_specs=[
            pl.BlockSpec((TC_BLK, D), lambda i: (i, 0)),
            pl.BlockSpec((1, 1, TC_BLK), lambda i: (i, 0, 0)),
            pl.BlockSpec((TABLE_PAD, D), lambda i: (0, 0)),
        ],
        out_specs=pl.BlockSpec((TC_BLK, D), lambda i: (i, 0)),
    )(emb, ids_blk, table_pad)


def kernel(batch_Phrase_emb, Phrase_type_ids, phrase_attribute_emb_all):
    B, D = batch_Phrase_emb.shape
    tc_rows = B - SC_ROWS

    sc_out = _sc_part(batch_Phrase_emb, Phrase_type_ids[tc_rows:],
                      phrase_attribute_emb_all, tc_rows, SC_ROWS)
    tc_out = _tc_part(batch_Phrase_emb, Phrase_type_ids[:tc_rows],
                      phrase_attribute_emb_all, tc_rows)
    return lax.dynamic_update_slice(tc_out, sc_out, (tc_rows, 0))
